# baseline (speedup 1.0000x reference)
import functools

import jax
import jax.numpy as jnp
from jax import lax
from jax.experimental import pallas as pl
from jax.experimental.pallas import tpu as pltpu, tpu_sc as plsc

VOCAB = 1_000_000
D = 64
IDX_BLK = 128
K = 8
S = K * IDX_BLK
NC = 2
NS = 16
NW = NC * NS


def _make_gather(n_rows: int):
    n_blocks = n_rows // IDX_BLK
    blocks_per_w = n_blocks // NW
    slabs_per_w = blocks_per_w // K

    mesh = plsc.VectorSubcoreMesh(core_axis_name="c", subcore_axis_name="s")

    @functools.partial(
        pl.kernel,
        mesh=mesh,
        out_type=jax.ShapeDtypeStruct((n_rows, D), jnp.float32),
        scratch_types=[
            pltpu.VMEM((K, IDX_BLK), jnp.int32),
            pltpu.VMEM((S, D), jnp.float32),
            pltpu.SemaphoreType.DMA,
        ],
        compiler_params=pltpu.CompilerParams(use_tc_tiling_on_sc=False),
    )
    def gather_kernel(table_hbm, idx_hbm, out_hbm, idx_v, rows_v, sem):
        wid = lax.axis_index("s") * NC + lax.axis_index("c")

        def slab(i, carry):
            blk0 = wid * blocks_per_w + i * K
            pltpu.sync_copy(idx_hbm.at[pl.ds(blk0, K)], idx_v)
            descs = [
                pltpu.async_copy(
                    table_hbm.at[idx_v.at[j]],
                    rows_v.at[pl.ds(j * IDX_BLK, IDX_BLK)],
                    sem,
                )
                for j in range(K)
            ]
            for d in descs:
                d.wait()
            pltpu.sync_copy(rows_v, out_hbm.at[pl.ds(blk0 * IDX_BLK, S)])
            return carry

        lax.fori_loop(0, slabs_per_w, slab, 0)

    return gather_kernel


def kernel(inputs, weight):
    b, t = inputs.shape
    n_rows = b * t
    idx = inputs.reshape(n_rows // IDX_BLK, IDX_BLK).astype(jnp.int32)
    out = _make_gather(n_rows)(weight, idx)
    return out.reshape(b, t, D)


# speedup vs baseline: 1.0203x; 1.0203x over previous
import functools

import jax
import jax.numpy as jnp
from jax import lax
from jax.experimental import pallas as pl
from jax.experimental.pallas import tpu as pltpu, tpu_sc as plsc

D = 64
IDX_BLK = 128
K = 5
S = K * IDX_BLK
NC = 2
NS = 16
NW = NC * NS


def _make_gather(n_rows: int):
    n_blocks = n_rows // IDX_BLK
    blocks_per_w = n_blocks // NW
    nslabs = blocks_per_w // K
    npairs = nslabs // 2

    mesh = plsc.VectorSubcoreMesh(core_axis_name="c", subcore_axis_name="s")

    @functools.partial(
        pl.kernel,
        mesh=mesh,
        out_type=jax.ShapeDtypeStruct((n_rows, D), jnp.float32),
        scratch_types=[
            pltpu.VMEM((blocks_per_w, IDX_BLK), jnp.int32),
            pltpu.VMEM((S, D), jnp.float32),
            pltpu.VMEM((S, D), jnp.float32),
            pltpu.SemaphoreType.DMA,
            pltpu.SemaphoreType.DMA,
            pltpu.SemaphoreType.DMA,
            pltpu.SemaphoreType.DMA,
        ],
        compiler_params=pltpu.CompilerParams(use_tc_tiling_on_sc=False),
    )
    def gather_kernel(table_hbm, idx_hbm, out_hbm,
                      idx_v, rows0, rows1, g0, g1, w0, w1):
        wid = lax.axis_index("s") * NC + lax.axis_index("c")
        blk_base = wid * blocks_per_w
        rows = (rows0, rows1)
        gsem = (g0, g1)
        wsem = (w0, w1)

        pltpu.sync_copy(idx_hbm.at[pl.ds(blk_base, blocks_per_w)], idx_v)

        def fire_g(slab, b):
            for j in range(K):
                pltpu.async_copy(
                    table_hbm.at[idx_v.at[slab * K + j]],
                    rows[b].at[pl.ds(j * IDX_BLK, IDX_BLK)],
                    gsem[b],
                )

        def drain_g(b):
            pltpu.make_async_copy(table_hbm.at[pl.ds(0, S)], rows[b],
                                  gsem[b]).wait()

        def fire_w(slab, b):
            dst = out_hbm.at[pl.ds((blk_base + slab * K) * IDX_BLK, S)]
            pltpu.async_copy(rows[b], dst, wsem[b])

        def drain_w(b):
            pltpu.make_async_copy(rows[b], out_hbm.at[pl.ds(0, S)],
                                  wsem[b]).wait()

        fire_g(0, 0)

        def pair(p, carry):
            i = 2 * p
            @pl.when(p >= 1)
            def _():
                drain_w(1)
            fire_g(i + 1, 1)
            drain_g(0)
            fire_w(i, 0)

            @pl.when(p <= npairs - 2)
            def _():
                drain_w(0)
                fire_g(i + 2, 0)
            drain_g(1)
            fire_w(i + 1, 1)
            return carry

        lax.fori_loop(0, npairs, pair, 0)
        drain_w(0)
        drain_w(1)

    return gather_kernel


def kernel(inputs, weight):
    b, t = inputs.shape
    n_rows = b * t
    idx = inputs.reshape(n_rows // IDX_BLK, IDX_BLK).astype(jnp.int32)
    out = _make_gather(n_rows)(weight, idx)
    return out.reshape(b, t, D)
